# Initial kernel scaffold; baseline (speedup 1.0000x reference)
import jax
import jax.numpy as jnp
from jax.experimental import pallas as pl


def kernel(x, row_splits, W1_s, b1_s, W1_h, b1_h, W1_o, b1_o, W2_s, b2_s, W2_h, b2_h, W2_o, b2_o, Wb1, bb1, Wb2, bb2, Wb3, bb3):
    raise NotImplementedError("write your pallas kernel here")


# speedup vs baseline: 7.5002x; 7.5002x over previous
import functools

import jax
import jax.numpy as jnp
from jax import lax
from jax.experimental import pallas as pl
from jax.experimental.pallas import tpu as pltpu
from jax.experimental.pallas import tpu_sc as plsc

N = 10000
D = 128
S = 8
F = 8
K = 16
NPAD = 10240
ROW_TILE = 256
GRID_ROWS = (N + ROW_TILE - 1) // ROW_TILE
NW = 32
JSLOTS = 384
OUT_PAD = 10016
DUMMY_ROW = OUT_PAD - 1
BIG = 3.0e38


def _proj_body(x_ref, w_ref, b_ref, o_ref):
    o_ref[...] = (
        jnp.dot(x_ref[...], w_ref[...], preferred_element_type=jnp.float32)
        + b_ref[...]
    )


def _proj(x, w, b2d):
    din, dout = w.shape
    return pl.pallas_call(
        _proj_body,
        grid=(GRID_ROWS,),
        in_specs=[
            pl.BlockSpec((ROW_TILE, din), lambda i: (i, 0)),
            pl.BlockSpec((din, dout), lambda i: (0, 0)),
            pl.BlockSpec((1, dout), lambda i: (0, 0)),
        ],
        out_specs=pl.BlockSpec((ROW_TILE, dout), lambda i: (i, 0)),
        out_shape=jax.ShapeDtypeStruct((N, dout), jnp.float32),
    )(x, w, b2d)


def _mid_body(xin_ref, agg_ref, wox_ref, woa_ref, bo_ref, wsh_ref, bsh_ref,
              lat_ref, sh_ref):
    lat = (
        jnp.dot(xin_ref[...], wox_ref[...], preferred_element_type=jnp.float32)
        + jnp.dot(agg_ref[...], woa_ref[...], preferred_element_type=jnp.float32)
        + bo_ref[...]
    )
    lat_ref[...] = lat
    sh_ref[...] = (
        jnp.dot(lat, wsh_ref[...], preferred_element_type=jnp.float32)
        + bsh_ref[...]
    )


def _mid(xin, agg, wox, woa, bo2d, wsh, bsh2d):
    return pl.pallas_call(
        _mid_body,
        grid=(GRID_ROWS,),
        in_specs=[
            pl.BlockSpec((ROW_TILE, D), lambda i: (i, 0)),
            pl.BlockSpec((ROW_TILE, 2 * F), lambda i: (i, 0)),
            pl.BlockSpec((D, D), lambda i: (0, 0)),
            pl.BlockSpec((2 * F, D), lambda i: (0, 0)),
            pl.BlockSpec((1, D), lambda i: (0, 0)),
            pl.BlockSpec((D, 2 * F), lambda i: (0, 0)),
            pl.BlockSpec((1, 2 * F), lambda i: (0, 0)),
        ],
        out_specs=[
            pl.BlockSpec((ROW_TILE, D), lambda i: (i, 0)),
            pl.BlockSpec((ROW_TILE, 2 * F), lambda i: (i, 0)),
        ],
        out_shape=[
            jax.ShapeDtypeStruct((N, D), jnp.float32),
            jax.ShapeDtypeStruct((N, 2 * F), jnp.float32),
        ],
    )(xin, agg, wox, woa, bo2d, wsh, bsh2d)


def _final_body(xin_ref, agg_ref, wox_ref, woa_ref, bo_ref,
                wb1_ref, bb1_ref, wb2_ref, bb2_ref, wb3_ref, bb3_ref,
                beta_ref, lat_ref):
    lat = (
        jnp.dot(xin_ref[...], wox_ref[...], preferred_element_type=jnp.float32)
        + jnp.dot(agg_ref[...], woa_ref[...], preferred_element_type=jnp.float32)
        + bo_ref[...]
    )
    lat_ref[...] = lat
    hb = jax.nn.relu(
        jnp.dot(lat, wb1_ref[...], preferred_element_type=jnp.float32)
        + bb1_ref[...]
    )
    hb = jax.nn.relu(
        jnp.dot(hb, wb2_ref[...], preferred_element_type=jnp.float32)
        + bb2_ref[...]
    )
    logit = (
        jnp.dot(hb, wb3_ref[...], preferred_element_type=jnp.float32)
        + bb3_ref[...]
    )
    beta = jax.nn.sigmoid(logit)
    beta_ref[...] = jnp.clip(beta, 1e-6, 1.0 - 1e-6)


def _final(xin, agg, wox, woa, bo2d, wb1, bb1_2d, wb2, bb2_2d, wb3, bb3_2d):
    return pl.pallas_call(
        _final_body,
        grid=(GRID_ROWS,),
        in_specs=[
            pl.BlockSpec((ROW_TILE, D), lambda i: (i, 0)),
            pl.BlockSpec((ROW_TILE, 2 * F), lambda i: (i, 0)),
            pl.BlockSpec((D, D), lambda i: (0, 0)),
            pl.BlockSpec((2 * F, D), lambda i: (0, 0)),
            pl.BlockSpec((1, D), lambda i: (0, 0)),
            pl.BlockSpec((D, D // 2), lambda i: (0, 0)),
            pl.BlockSpec((1, D // 2), lambda i: (0, 0)),
            pl.BlockSpec((D // 2, D // 4), lambda i: (0, 0)),
            pl.BlockSpec((1, D // 4), lambda i: (0, 0)),
            pl.BlockSpec((D // 4, 1), lambda i: (0, 0)),
            pl.BlockSpec((1, 1), lambda i: (0, 0)),
        ],
        out_specs=[
            pl.BlockSpec((ROW_TILE, 1), lambda i: (i, 0)),
            pl.BlockSpec((ROW_TILE, D), lambda i: (i, 0)),
        ],
        out_shape=[
            jax.ShapeDtypeStruct((N, 1), jnp.float32),
            jax.ShapeDtypeStruct((N, D), jnp.float32),
        ],
    )(xin, agg, wox, woa, bo2d, wb1, bb1_2d, wb2, bb2_2d, wb3, bb3_2d)


def _sc_knn(sT, sh, splits16):
    mesh = plsc.VectorSubcoreMesh(core_axis_name="c", subcore_axis_name="s")

    @functools.partial(
        pl.kernel,
        mesh=mesh,
        compiler_params=pltpu.CompilerParams(
            needs_layout_passes=False, use_tc_tiling_on_sc=False),
        out_type=jax.ShapeDtypeStruct((OUT_PAD, 2 * F), jnp.float32),
        scratch_types=[
            pltpu.VMEM((S, NPAD), jnp.float32),
            pltpu.VMEM((16,), jnp.int32),
            pltpu.VMEM((JSLOTS, 2 * F), jnp.float32),
            pltpu.VMEM((JSLOTS // 128, 128), jnp.int32),
            pltpu.VMEM((K, 2 * F), jnp.float32),
            pltpu.VMEM((32,), jnp.float32),
            pltpu.SemaphoreType.DMA,
        ],
    )
    def knn_kernel(sT_hbm, sh_hbm, splits_hbm, out_hbm,
                   sT_v, splits_v, out_v, rowid_v, nbr_v, atmp_v, sem):
        wid = lax.axis_index("s") * 2 + lax.axis_index("c")
        pltpu.sync_copy(sT_hbm, sT_v)
        pltpu.sync_copy(splits_hbm, splits_v)

        lanes = lax.iota(jnp.int32, 16)
        sel_idx = jnp.where(lanes < F, lanes + F, lanes + 2 * F)

        for ci in range(JSLOTS // 128):
            for q in range(8):
                r_ids = wid + NW * (ci * 128 + q * 16 + lanes)
                rowid_v[ci, pl.ds(q * 16, 16)] = jnp.where(
                    r_ids < N, r_ids, DUMMY_ROW)

        def row_body(j, carry):
            r = wid + NW * j

            @pl.when(r < N)
            def _():
                splits = splits_v[...]
                le = splits <= r
                spf = splits.astype(jnp.float32)
                lov = jnp.where(le, spf, 0.0)
                hiv = jnp.where(le, float(N), spf)
                lo = plsc.sort_key_val(lov, lov)[0][15].astype(jnp.int32)
                hi = plsc.sort_key_val(hiv, hiv)[0][0].astype(jnp.int32)
                srow = [sT_v[d, pl.ds(r, 16)][0] for d in range(S)]
                lo_al = (lo // 16) * 16
                nch = (hi - lo_al + 15) // 16

                def chunk_body(ci, carry):
                    tv, tix, worst = carry
                    c = lo_al + ci * 16
                    cols = c + lanes
                    acc = jnp.zeros((16,), jnp.float32)
                    for d in range(S):
                        diff = sT_v[d, pl.ds(c, 16)] - srow[d]
                        acc = acc + diff * diff
                    ok = (cols >= lo) & (cols < hi)
                    acc = jnp.where(ok, acc, BIG)

                    def merge(args):
                        tv, tix, _ = args
                        sk, sv = plsc.sort_key_val(acc, cols)
                        rk = lax.rev(sk, (0,))
                        rv = lax.rev(sv, (0,))
                        keep = tv <= rk
                        nv = jnp.where(keep, tv, rk)
                        ni = jnp.where(keep, tix, rv)
                        nv2, ni2 = plsc.sort_key_val(nv, ni)
                        return nv2, ni2, nv2[15]

                    return lax.cond(jnp.any(acc < worst), merge,
                                    lambda args: args, (tv, tix, worst))

                tv0 = jnp.full((16,), BIG, jnp.float32)
                ti0 = jnp.zeros((16,), jnp.int32)
                tv, tix, _ = lax.fori_loop(
                    0, nch, chunk_body, (tv0, ti0, jnp.float32(BIG)))

                pltpu.async_copy(sh_hbm.at[tix], nbr_v, sem).wait()
                w = jnp.exp(tv * -10.0)
                macc = jnp.zeros((16,), jnp.float32)
                mmax = jnp.full((16,), -BIG, jnp.float32)
                for k in range(K):
                    msg = nbr_v[k, :] * w[k]
                    macc = macc + msg
                    mmax = jnp.maximum(mmax, msg)
                atmp_v[pl.ds(0, 16)] = macc * (1.0 / K)
                atmp_v[pl.ds(16, 16)] = mmax
                out_v[j, :] = plsc.load_gather(atmp_v, [sel_idx])

            return carry

        lax.fori_loop(0, JSLOTS, row_body, 0)

        for ci in range(JSLOTS // 128):
            pltpu.async_copy(
                out_v.at[pl.ds(ci * 128, 128)],
                out_hbm.at[rowid_v.at[ci]],
                sem,
            ).wait()

    return knn_kernel(sT, sh, splits16)


def _pad_t(sh):
    return jnp.pad(sh[:, :S].T, ((0, 0), (0, NPAD - N)))


def kernel(x, row_splits, W1_s, b1_s, W1_h, b1_h, W1_o, b1_o,
           W2_s, b2_s, W2_h, b2_h, W2_o, b2_o,
           Wb1, bb1, Wb2, bb2, Wb3, bb3):
    splits16 = jnp.concatenate(
        [row_splits.astype(jnp.int32),
         jnp.full((16 - row_splits.shape[0],), N, jnp.int32)])

    w1sh = jnp.concatenate([W1_s, W1_h], axis=1)
    b1sh = jnp.concatenate([b1_s, b1_h]).reshape(1, 2 * F)
    sh1 = _proj(x, w1sh, b1sh)
    agg1 = _sc_knn(_pad_t(sh1), sh1, splits16)[:N]

    w2sh = jnp.concatenate([W2_s, W2_h], axis=1)
    b2sh = jnp.concatenate([b2_s, b2_h]).reshape(1, 2 * F)
    latent1, sh2 = _mid(x, agg1, W1_o[:D], W1_o[D:], b1_o.reshape(1, D),
                        w2sh, b2sh)
    agg2 = _sc_knn(_pad_t(sh2), sh2, splits16)[:N]

    beta, latent2 = _final(latent1, agg2, W2_o[:D], W2_o[D:],
                           b2_o.reshape(1, D),
                           Wb1, bb1.reshape(1, D // 2),
                           Wb2, bb2.reshape(1, D // 4),
                           Wb3, bb3.reshape(1, 1))
    return beta, latent2


# speedup vs baseline: 17.9150x; 2.3886x over previous
import functools

import jax
import jax.numpy as jnp
from jax import lax
from jax.experimental import pallas as pl
from jax.experimental.pallas import tpu as pltpu
from jax.experimental.pallas import tpu_sc as plsc

N = 10000
D = 128
S = 8
F = 8
K = 16
NPAD = 10240
ROW_TILE = 256
GRID_ROWS = (N + ROW_TILE - 1) // ROW_TILE
NW = 32
GRP = 4
JSLOTS = 384
OUT_PAD = 10016
DUMMY_ROW = OUT_PAD - 1
BIG = 3.0e38


def _proj_body(x_ref, w_ref, b_ref, o_ref):
    o_ref[...] = (
        jnp.dot(x_ref[...], w_ref[...], preferred_element_type=jnp.float32)
        + b_ref[...]
    )


def _proj(x, w, b2d):
    din, dout = w.shape
    return pl.pallas_call(
        _proj_body,
        grid=(GRID_ROWS,),
        in_specs=[
            pl.BlockSpec((ROW_TILE, din), lambda i: (i, 0)),
            pl.BlockSpec((din, dout), lambda i: (0, 0)),
            pl.BlockSpec((1, dout), lambda i: (0, 0)),
        ],
        out_specs=pl.BlockSpec((ROW_TILE, dout), lambda i: (i, 0)),
        out_shape=jax.ShapeDtypeStruct((N, dout), jnp.float32),
    )(x, w, b2d)


def _mid_body(xin_ref, agg_ref, wox_ref, woa_ref, bo_ref, wsh_ref, bsh_ref,
              lat_ref, sh_ref):
    lat = (
        jnp.dot(xin_ref[...], wox_ref[...], preferred_element_type=jnp.float32)
        + jnp.dot(agg_ref[...], woa_ref[...], preferred_element_type=jnp.float32)
        + bo_ref[...]
    )
    lat_ref[...] = lat
    sh_ref[...] = (
        jnp.dot(lat, wsh_ref[...], preferred_element_type=jnp.float32)
        + bsh_ref[...]
    )


def _mid(xin, agg, wox, woa, bo2d, wsh, bsh2d):
    return pl.pallas_call(
        _mid_body,
        grid=(GRID_ROWS,),
        in_specs=[
            pl.BlockSpec((ROW_TILE, D), lambda i: (i, 0)),
            pl.BlockSpec((ROW_TILE, 2 * F), lambda i: (i, 0)),
            pl.BlockSpec((D, D), lambda i: (0, 0)),
            pl.BlockSpec((2 * F, D), lambda i: (0, 0)),
            pl.BlockSpec((1, D), lambda i: (0, 0)),
            pl.BlockSpec((D, 2 * F), lambda i: (0, 0)),
            pl.BlockSpec((1, 2 * F), lambda i: (0, 0)),
        ],
        out_specs=[
            pl.BlockSpec((ROW_TILE, D), lambda i: (i, 0)),
            pl.BlockSpec((ROW_TILE, 2 * F), lambda i: (i, 0)),
        ],
        out_shape=[
            jax.ShapeDtypeStruct((N, D), jnp.float32),
            jax.ShapeDtypeStruct((N, 2 * F), jnp.float32),
        ],
    )(xin, agg, wox, woa, bo2d, wsh, bsh2d)


def _final_body(xin_ref, agg_ref, wox_ref, woa_ref, bo_ref,
                wb1_ref, bb1_ref, wb2_ref, bb2_ref, wb3_ref, bb3_ref,
                beta_ref, lat_ref):
    lat = (
        jnp.dot(xin_ref[...], wox_ref[...], preferred_element_type=jnp.float32)
        + jnp.dot(agg_ref[...], woa_ref[...], preferred_element_type=jnp.float32)
        + bo_ref[...]
    )
    lat_ref[...] = lat
    hb = jax.nn.relu(
        jnp.dot(lat, wb1_ref[...], preferred_element_type=jnp.float32)
        + bb1_ref[...]
    )
    hb = jax.nn.relu(
        jnp.dot(hb, wb2_ref[...], preferred_element_type=jnp.float32)
        + bb2_ref[...]
    )
    logit = (
        jnp.dot(hb, wb3_ref[...], preferred_element_type=jnp.float32)
        + bb3_ref[...]
    )
    beta = jax.nn.sigmoid(logit)
    beta_ref[...] = jnp.clip(beta, 1e-6, 1.0 - 1e-6)


def _final(xin, agg, wox, woa, bo2d, wb1, bb1_2d, wb2, bb2_2d, wb3, bb3_2d):
    return pl.pallas_call(
        _final_body,
        grid=(GRID_ROWS,),
        in_specs=[
            pl.BlockSpec((ROW_TILE, D), lambda i: (i, 0)),
            pl.BlockSpec((ROW_TILE, 2 * F), lambda i: (i, 0)),
            pl.BlockSpec((D, D), lambda i: (0, 0)),
            pl.BlockSpec((2 * F, D), lambda i: (0, 0)),
            pl.BlockSpec((1, D), lambda i: (0, 0)),
            pl.BlockSpec((D, D // 2), lambda i: (0, 0)),
            pl.BlockSpec((1, D // 2), lambda i: (0, 0)),
            pl.BlockSpec((D // 2, D // 4), lambda i: (0, 0)),
            pl.BlockSpec((1, D // 4), lambda i: (0, 0)),
            pl.BlockSpec((D // 4, 1), lambda i: (0, 0)),
            pl.BlockSpec((1, 1), lambda i: (0, 0)),
        ],
        out_specs=[
            pl.BlockSpec((ROW_TILE, 1), lambda i: (i, 0)),
            pl.BlockSpec((ROW_TILE, D), lambda i: (i, 0)),
        ],
        out_shape=[
            jax.ShapeDtypeStruct((N, 1), jnp.float32),
            jax.ShapeDtypeStruct((N, D), jnp.float32),
        ],
    )(xin, agg, wox, woa, bo2d, wb1, bb1_2d, wb2, bb2_2d, wb3, bb3_2d)


def _sc_knn(sT, sh, splits16):
    mesh = plsc.VectorSubcoreMesh(core_axis_name="c", subcore_axis_name="s")

    @functools.partial(
        pl.kernel,
        mesh=mesh,
        compiler_params=pltpu.CompilerParams(
            needs_layout_passes=False, use_tc_tiling_on_sc=False),
        out_type=jax.ShapeDtypeStruct((OUT_PAD, 2 * F), jnp.float32),
        scratch_types=[
            pltpu.VMEM((S, NPAD), jnp.float32),
            pltpu.VMEM((16,), jnp.int32),
            pltpu.VMEM((JSLOTS, 2 * F), jnp.float32),
            pltpu.VMEM((JSLOTS // 128, 128), jnp.int32),
            pltpu.VMEM((GRP, K, 2 * F), jnp.float32),
            pltpu.VMEM((32,), jnp.float32),
            pltpu.SemaphoreType.DMA,
        ],
    )
    def knn_kernel(sT_hbm, sh_hbm, splits_hbm, out_hbm,
                   sT_v, splits_v, out_v, rowid_v, nbr_v, atmp_v, sem):
        wid = lax.axis_index("s") * 2 + lax.axis_index("c")
        pltpu.sync_copy(sT_hbm, sT_v)
        pltpu.sync_copy(splits_hbm, splits_v)

        lanes = lax.iota(jnp.int32, 16)
        sel_idx = jnp.where(lanes < F, lanes + F, lanes + 2 * F)

        for ci in range(JSLOTS // 128):
            for q in range(8):
                r_ids = wid + NW * (ci * 128 + q * 16 + lanes)
                rowid_v[ci, pl.ds(q * 16, 16)] = jnp.where(
                    r_ids < N, r_ids, DUMMY_ROW)

        def group_body(g, carry):
            j0 = g * GRP
            splits = splits_v[...]
            spf = splits.astype(jnp.float32)
            los, his, srows = [], [], []
            for u in range(GRP):
                r_u = wid + NW * (j0 + u)
                vrow = r_u < N
                le = splits <= r_u
                lov = jnp.where(le, spf, 0.0)
                hiv = jnp.where(le, float(N), spf)
                lo_u = plsc.sort_key_val(lov, lov)[0][15].astype(jnp.int32)
                hi_u = plsc.sort_key_val(hiv, hiv)[0][0].astype(jnp.int32)
                los.append(jnp.where(vrow, lo_u, N))
                his.append(jnp.where(vrow, hi_u, 0))
                rs = jnp.minimum(r_u, N - 1)
                srows.append([sT_v[d, pl.ds(rs, 16)][0] for d in range(S)])
            glo = jnp.minimum(jnp.minimum(los[0], los[1]),
                              jnp.minimum(los[2], los[3]))
            ghi = jnp.maximum(jnp.maximum(his[0], his[1]),
                              jnp.maximum(his[2], his[3]))
            glo_al = (glo // 16) * 16
            nch = jnp.maximum(0, (ghi - glo_al + 15) // 16)

            def chunk_body(ci, carry):
                c = glo_al + ci * 16
                cols = c + lanes
                vs = [sT_v[d, pl.ds(c, 16)] for d in range(S)]
                out = []
                for u in range(GRP):
                    tv, tix, worst = carry[3 * u], carry[3 * u + 1], carry[3 * u + 2]
                    acc = jnp.zeros((16,), jnp.float32)
                    for d in range(S):
                        diff = vs[d] - srows[u][d]
                        acc = acc + diff * diff
                    acc = jnp.where((cols >= los[u]) & (cols < his[u]),
                                    acc, BIG)

                    def merge(args, acc=acc, cols=cols):
                        tv, tix, _ = args
                        sk, sv = plsc.sort_key_val(acc, cols)
                        rk = lax.rev(sk, (0,))
                        rv = lax.rev(sv, (0,))
                        keep = tv <= rk
                        nv = jnp.where(keep, tv, rk)
                        ni = jnp.where(keep, tix, rv)
                        nv2, ni2 = plsc.sort_key_val(nv, ni)
                        return nv2, ni2, nv2[15]

                    tv, tix, worst = lax.cond(
                        jnp.any(acc < worst), merge,
                        lambda args: args, (tv, tix, worst))
                    out += [tv, tix, worst]
                return tuple(out)

            init = []
            for u in range(GRP):
                init += [jnp.full((16,), BIG, jnp.float32),
                         jnp.zeros((16,), jnp.int32), jnp.float32(BIG)]
            res = lax.fori_loop(0, nch, chunk_body, tuple(init))

            cps = [pltpu.async_copy(sh_hbm.at[res[3 * u + 1]], nbr_v.at[u],
                                    sem)
                   for u in range(GRP)]
            for u in range(GRP):
                cps[u].wait()
                w = jnp.exp(res[3 * u] * -10.0)
                macc = jnp.zeros((16,), jnp.float32)
                mmax = jnp.full((16,), -BIG, jnp.float32)
                for k in range(K):
                    msg = nbr_v[u, k, :] * w[k]
                    macc = macc + msg
                    mmax = jnp.maximum(mmax, msg)
                atmp_v[pl.ds(0, 16)] = macc * (1.0 / K)
                atmp_v[pl.ds(16, 16)] = mmax
                out_v[j0 + u, :] = plsc.load_gather(atmp_v, [sel_idx])

            return carry

        lax.fori_loop(0, JSLOTS // GRP, group_body, 0)

        for ci in range(JSLOTS // 128):
            pltpu.async_copy(
                out_v.at[pl.ds(ci * 128, 128)],
                out_hbm.at[rowid_v.at[ci]],
                sem,
            ).wait()

    return knn_kernel(sT, sh, splits16)


def _pad_t(sh):
    return jnp.pad(sh[:, :S].T, ((0, 0), (0, NPAD - N)))


def kernel(x, row_splits, W1_s, b1_s, W1_h, b1_h, W1_o, b1_o,
           W2_s, b2_s, W2_h, b2_h, W2_o, b2_o,
           Wb1, bb1, Wb2, bb2, Wb3, bb3):
    splits16 = jnp.concatenate(
        [row_splits.astype(jnp.int32),
         jnp.full((16 - row_splits.shape[0],), N, jnp.int32)])

    w1sh = jnp.concatenate([W1_s, W1_h], axis=1)
    b1sh = jnp.concatenate([b1_s, b1_h]).reshape(1, 2 * F)
    sh1 = _proj(x, w1sh, b1sh)
    agg1 = _sc_knn(_pad_t(sh1), sh1, splits16)[:N]

    w2sh = jnp.concatenate([W2_s, W2_h], axis=1)
    b2sh = jnp.concatenate([b2_s, b2_h]).reshape(1, 2 * F)
    latent1, sh2 = _mid(x, agg1, W1_o[:D], W1_o[D:], b1_o.reshape(1, D),
                        w2sh, b2sh)
    agg2 = _sc_knn(_pad_t(sh2), sh2, splits16)[:N]

    beta, latent2 = _final(latent1, agg2, W2_o[:D], W2_o[D:],
                           b2_o.reshape(1, D),
                           Wb1, bb1.reshape(1, D // 2),
                           Wb2, bb2.reshape(1, D // 4),
                           Wb3, bb3.reshape(1, 1))
    return beta, latent2
